# Initial kernel scaffold; baseline (speedup 1.0000x reference)
import jax
import jax.numpy as jnp
from jax.experimental import pallas as pl


def kernel(template, projections):
    raise NotImplementedError("write your pallas kernel here")


# speedup vs baseline: 311.2176x; 311.2176x over previous
import jax
import jax.numpy as jnp
from jax.experimental import pallas as pl

_V = 4096
_N = 10
_SUB = 8
_LANE = 512
_T = 40


def _bfround(a):
    u = jax.lax.bitcast_convert_type(a, jnp.uint32)
    u = u + jnp.uint32(0x7FFF) + ((u >> 16) & jnp.uint32(1))
    u = u & jnp.uint32(0xFFFF0000)
    return jax.lax.bitcast_convert_type(u, jnp.float32)


def _bc_kernel(tx_ref, ty_ref, px_ref, py_ref,
               w0_ref, w2_ref, w1_ref, i0_ref, i1_ref, i2_ref):
    f32 = jnp.float32
    tx = tx_ref[0]
    ty = ty_ref[0]

    d = []
    nid = []
    px = []
    py = []
    for k in range(_N):
        xk = px_ref[k]
        yk = py_ref[k]
        dxk = tx - xk
        dyk = ty - yk
        d.append(jnp.sqrt(dxk * dxk + dyk * dyk))
        nid.append(jnp.full((_SUB, _LANE), k, jnp.int32))
        px.append(xk)
        py.append(yk)

    for i in range(1, _N):
        for j in range(i, 0, -1):
            a, b = j - 1, j
            swap = (d[b] < d[a]) | ((d[b] == d[a]) & (nid[b] < nid[a]))
            for arr in (d, nid, px, py):
                ai, bi = arr[a], arr[b]
                arr[a] = jnp.where(swap, bi, ai)
                arr[b] = jnp.where(swap, ai, bi)

    wx = tx - px[0]
    wy = ty - py[0]
    wxb = _bfround(wx)
    wyb = _bfround(wy)
    ex = [px[n] - px[0] for n in range(1, _N)]
    ey = [py[n] - py[0] for n in range(1, _N)]
    d00 = [ex[n] * ex[n] + ey[n] * ey[n] for n in range(9)]
    bx = [_bfround(ex[n]) for n in range(9)]
    by = [_bfround(ey[n]) for n in range(9)]
    d02 = [bx[n] * wxb + by[n] * wyb for n in range(9)]

    inf = jnp.full((_SUB, _LANE), jnp.inf, f32)
    one = jnp.float32(1.0)
    best = inf
    bw0, bw2, bw1 = one - inf, inf, inf
    bi1 = nid[1]
    bi2 = nid[1]
    first = True
    for n in range(9):
        for m in range(9):
            d01 = bx[n] * bx[m] + by[n] * by[m]
            den = d00[n] * d00[m] - d01 * d01
            den = jnp.where(den == 0.0, jnp.float32(1e-10), den)
            p2 = (d00[m] * d02[n] - d01 * d02[m]) / den
            p1 = (d00[n] * d02[m] - d01 * d02[n]) / den
            p0 = (one - p2) - p1
            a0 = jnp.where(p0 <= 0.0, inf, p0)
            a2 = jnp.where(p2 <= 0.0, inf, p2)
            a1 = jnp.where(p1 <= 0.0, inf, p1)
            sc = jnp.maximum(jnp.maximum(a0 * a0, a2 * a2), a1 * a1)
            if first:
                best, bw0, bw2, bw1 = sc, a0, a2, a1
                first = False
                continue
            upd = (sc < best) | (jnp.isnan(sc) & ~jnp.isnan(best))
            best = jnp.where(upd, sc, best)
            bw0 = jnp.where(upd, a0, bw0)
            bw2 = jnp.where(upd, a2, bw2)
            bw1 = jnp.where(upd, a1, bw1)
            bi1 = jnp.where(upd, nid[n + 1], bi1)
            bi2 = jnp.where(upd, nid[m + 1], bi2)

    has_inf = jnp.isinf(bw0) | jnp.isinf(bw2) | jnp.isinf(bw1)
    zero = jnp.float32(0.0)
    w0_ref[0] = jnp.where(has_inf, zero, bw0)
    w2_ref[0] = jnp.where(has_inf, zero, bw2)
    w1_ref[0] = jnp.where(has_inf, zero, bw1)
    i0_ref[0] = nid[0]
    i1_ref[0] = bi1
    i2_ref[0] = bi2


def kernel(template, projections):
    R, A = template.shape[0], template.shape[1]
    f32 = jnp.float32
    t = template.astype(f32).reshape(_T, 2)
    proj = projections.astype(f32)

    px = proj[:, :, 0].T.reshape(_N, _SUB, _LANE)
    py = proj[:, :, 1].T.reshape(_N, _SUB, _LANE)
    tx = jnp.broadcast_to(t[:, 0][:, None, None], (_T, _SUB, _LANE))
    ty = jnp.broadcast_to(t[:, 1][:, None, None], (_T, _SUB, _LANE))

    plane = (1, _SUB, _LANE)
    full = (_N, _SUB, _LANE)
    t_spec = pl.BlockSpec(plane, lambda i: (i, 0, 0))
    p_spec = pl.BlockSpec(full, lambda i: (0, 0, 0))
    o_spec = pl.BlockSpec(plane, lambda i: (i, 0, 0))
    shape = jax.ShapeDtypeStruct((_T, _SUB, _LANE), f32)
    ishape = jax.ShapeDtypeStruct((_T, _SUB, _LANE), jnp.int32)

    w0, w2, w1, i0, i1, i2 = pl.pallas_call(
        _bc_kernel,
        grid=(_T,),
        in_specs=[t_spec, t_spec, p_spec, p_spec],
        out_specs=[o_spec] * 6,
        out_shape=[shape, shape, shape, ishape, ishape, ishape],
    )(tx, ty, px, py)

    w = jnp.stack([w0, w2, w1], axis=-1).reshape(_T, _V, 3)
    oi = jnp.stack([i0, i1, i2], axis=-1).reshape(_T, _V, 3)
    w = w.transpose(1, 0, 2).reshape(_V, R, A, 3)
    oi = oi.transpose(1, 0, 2).reshape(_V, R, A, 3)
    return w, oi


# speedup vs baseline: 327.9063x; 1.0536x over previous
import jax
import jax.numpy as jnp
from jax.experimental import pallas as pl

_V = 4096
_N = 10
_SUB = 8
_LANE = 512
_T = 40


def _bfround(a):
    u = jax.lax.bitcast_convert_type(a, jnp.uint32)
    u = u + jnp.uint32(0x7FFF) + ((u >> 16) & jnp.uint32(1))
    u = u & jnp.uint32(0xFFFF0000)
    return jax.lax.bitcast_convert_type(u, jnp.float32)


def _bc_kernel(tx_ref, ty_ref, px_ref, py_ref,
               w0_ref, w2_ref, w1_ref, i0_ref, i1_ref, i2_ref):
    f32 = jnp.float32
    tx = tx_ref[0]
    ty = ty_ref[0]

    d = []
    nid = []
    px = []
    py = []
    for k in range(_N):
        xk = px_ref[k]
        yk = py_ref[k]
        dxk = tx - xk
        dyk = ty - yk
        d.append(jnp.sqrt(dxk * dxk + dyk * dyk))
        nid.append(jnp.full((_SUB, _LANE), k, jnp.int32))
        px.append(xk)
        py.append(yk)

    for a, b in ((0, 5), (1, 6), (2, 7), (3, 8), (4, 9),
                 (0, 3), (1, 4), (5, 8), (6, 9),
                 (0, 2), (3, 6), (7, 9),
                 (0, 1), (2, 4), (5, 7), (8, 9),
                 (1, 2), (3, 5), (4, 6), (7, 8),
                 (1, 3), (2, 5), (4, 7), (6, 8),
                 (2, 3), (4, 5), (6, 7),
                 (3, 4), (5, 6)):
        swap = (d[b] < d[a]) | ((d[b] == d[a]) & (nid[b] < nid[a]))
        for arr in (d, nid, px, py):
            ai, bi = arr[a], arr[b]
            arr[a] = jnp.where(swap, bi, ai)
            arr[b] = jnp.where(swap, ai, bi)

    wx = tx - px[0]
    wy = ty - py[0]
    wxb = _bfround(wx)
    wyb = _bfround(wy)
    ex = [px[n] - px[0] for n in range(1, _N)]
    ey = [py[n] - py[0] for n in range(1, _N)]
    d00 = [ex[n] * ex[n] + ey[n] * ey[n] for n in range(9)]
    bx = [_bfround(ex[n]) for n in range(9)]
    by = [_bfround(ey[n]) for n in range(9)]
    d02 = [bx[n] * wxb + by[n] * wyb for n in range(9)]

    inf = jnp.full((_SUB, _LANE), jnp.inf, f32)
    one = jnp.float32(1.0)

    P2 = [[None] * 9 for _ in range(9)]
    P1 = [[None] * 9 for _ in range(9)]
    A2 = [[None] * 9 for _ in range(9)]
    A1 = [[None] * 9 for _ in range(9)]
    M21 = [[None] * 9 for _ in range(9)]
    for n in range(9):
        for m in range(n, 9):
            d01 = bx[n] * bx[m] + by[n] * by[m]
            den = d00[n] * d00[m] - d01 * d01
            den = jnp.where(den == 0.0, jnp.float32(1e-10), den)
            p2 = (d00[m] * d02[n] - d01 * d02[m]) / den
            p1 = (d00[n] * d02[m] - d01 * d02[n]) / den
            a2 = jnp.where(p2 <= 0.0, inf, p2)
            a1 = jnp.where(p1 <= 0.0, inf, p1)
            P2[n][m], P1[n][m] = p2, p1
            A2[n][m], A1[n][m] = a2, a1
            M21[n][m] = jnp.maximum(a2 * a2, a1 * a1)

    best = inf
    bw0, bw2, bw1 = inf, inf, inf
    bi1 = nid[1]
    bi2 = nid[1]
    first = True
    for n in range(9):
        for m in range(9):
            if n <= m:
                p2, p1 = P2[n][m], P1[n][m]
                a2, a1 = A2[n][m], A1[n][m]
                m21 = M21[n][m]
            else:
                p2, p1 = P1[m][n], P2[m][n]
                a2, a1 = A1[m][n], A2[m][n]
                m21 = M21[m][n]
            p0 = (one - p2) - p1
            a0 = jnp.where(p0 <= 0.0, inf, p0)
            sc = jnp.maximum(a0 * a0, m21)
            if first:
                best, bw0, bw2, bw1 = sc, a0, a2, a1
                first = False
                continue
            upd = (sc < best) | (jnp.isnan(sc) & ~jnp.isnan(best))
            best = jnp.where(upd, sc, best)
            bw0 = jnp.where(upd, a0, bw0)
            bw2 = jnp.where(upd, a2, bw2)
            bw1 = jnp.where(upd, a1, bw1)
            bi1 = jnp.where(upd, nid[n + 1], bi1)
            bi2 = jnp.where(upd, nid[m + 1], bi2)

    has_inf = jnp.isinf(bw0) | jnp.isinf(bw2) | jnp.isinf(bw1)
    zero = jnp.float32(0.0)
    w0_ref[0] = jnp.where(has_inf, zero, bw0)
    w2_ref[0] = jnp.where(has_inf, zero, bw2)
    w1_ref[0] = jnp.where(has_inf, zero, bw1)
    i0_ref[0] = nid[0]
    i1_ref[0] = bi1
    i2_ref[0] = bi2


def kernel(template, projections):
    R, A = template.shape[0], template.shape[1]
    f32 = jnp.float32
    t = template.astype(f32).reshape(_T, 2)
    proj = projections.astype(f32)

    px = proj[:, :, 0].T.reshape(_N, _SUB, _LANE)
    py = proj[:, :, 1].T.reshape(_N, _SUB, _LANE)
    tx = jnp.broadcast_to(t[:, 0][:, None, None], (_T, _SUB, _LANE))
    ty = jnp.broadcast_to(t[:, 1][:, None, None], (_T, _SUB, _LANE))

    plane = (1, _SUB, _LANE)
    full = (_N, _SUB, _LANE)
    t_spec = pl.BlockSpec(plane, lambda i: (i, 0, 0))
    p_spec = pl.BlockSpec(full, lambda i: (0, 0, 0))
    o_spec = pl.BlockSpec(plane, lambda i: (i, 0, 0))
    shape = jax.ShapeDtypeStruct((_T, _SUB, _LANE), f32)
    ishape = jax.ShapeDtypeStruct((_T, _SUB, _LANE), jnp.int32)

    w0, w2, w1, i0, i1, i2 = pl.pallas_call(
        _bc_kernel,
        grid=(_T,),
        in_specs=[t_spec, t_spec, p_spec, p_spec],
        out_specs=[o_spec] * 6,
        out_shape=[shape, shape, shape, ishape, ishape, ishape],
    )(tx, ty, px, py)

    w = jnp.stack([w0, w2, w1], axis=-1).reshape(_T, _V, 3)
    oi = jnp.stack([i0, i1, i2], axis=-1).reshape(_T, _V, 3)
    w = w.transpose(1, 0, 2).reshape(_V, R, A, 3)
    oi = oi.transpose(1, 0, 2).reshape(_V, R, A, 3)
    return w, oi
